# baseline (speedup 1.0000x reference)
import functools

import jax
import jax.numpy as jnp
from jax import lax
from jax.experimental import pallas as pl
from jax.experimental.pallas import tpu as pltpu
from jax.experimental.pallas import tpu_sc as plsc

NUM_SPEAKER = 100000
EMB_DIM = 64
BATCH = 4096

_A_LEN = 49920
_B_LEN = 50048
_T_OFF = _A_LEN + _B_LEN
_T_LEN = NUM_SPEAKER - _T_OFF
_TAIL_TOTAL = EMB_DIM * _T_LEN

_info = plsc.get_sparse_core_info()
_NC, _NS = _info.num_cores, _info.num_subcores
_NW = _NC * _NS
_ROWS_PER_W = EMB_DIM // _NW


@functools.partial(
    pl.kernel,
    mesh=plsc.VectorSubcoreMesh(core_axis_name="c", subcore_axis_name="s"),
    out_type=jax.ShapeDtypeStruct((EMB_DIM, BATCH), jnp.float32),
    scratch_types=[
        pltpu.VMEM((_A_LEN + _TAIL_TOTAL,), jnp.float32),
        pltpu.VMEM((_B_LEN,), jnp.float32),
        pltpu.VMEM((BATCH,), jnp.int32),
        pltpu.VMEM((BATCH,), jnp.float32),
        pltpu.SemaphoreType.DMA,
        pltpu.SemaphoreType.DMA,
    ],
    compiler_params=pltpu.CompilerParams(
        use_tc_tiling_on_sc=True, needs_layout_passes=False
    ),
)
def _lane_gather_kernel(
    tab_t, tail_flat, sid_hbm, out_t, buf_a, buf_b, sid_v, out_v, sem_a, sem_b
):
    wid = lax.axis_index("s") * _NC + lax.axis_index("c")
    c0 = wid * _ROWS_PER_W

    cp_a = pltpu.async_copy(
        tab_t.at[c0].at[pl.ds(0, _A_LEN)], buf_a.at[pl.ds(0, _A_LEN)], sem_a
    )
    cp_b = pltpu.async_copy(
        tab_t.at[c0].at[pl.ds(_A_LEN, _B_LEN)], buf_b, sem_b
    )
    pltpu.sync_copy(tail_flat, buf_a.at[pl.ds(_A_LEN, _TAIL_TOTAL)])
    pltpu.sync_copy(sid_hbm, sid_v)

    for r in range(_ROWS_PER_W):
        c = c0 + r
        tail_base = _A_LEN + c * _T_LEN - _T_OFF

        cp_a.wait()

        @plsc.parallel_loop(0, BATCH, step=16, unroll=4)
        def _(i):
            off = pl.multiple_of(i, 16)
            idx = sid_v[pl.ds(off, 16)]
            in_a = idx < _A_LEN
            in_t = idx >= _T_OFF
            mask = in_a | in_t
            rel = jnp.where(in_a, idx, idx + tail_base)
            rel = jnp.where(mask, rel, 0)
            vals = plsc.load_gather(buf_a, [rel], mask=mask)
            out_v[pl.ds(off, 16)] = jnp.where(mask, vals, 0.0)

        if r + 1 < _ROWS_PER_W:
            cp_a = pltpu.async_copy(
                tab_t.at[c + 1].at[pl.ds(0, _A_LEN)],
                buf_a.at[pl.ds(0, _A_LEN)],
                sem_a,
            )

        cp_b.wait()

        @plsc.parallel_loop(0, BATCH, step=16, unroll=4)
        def _(i):
            off = pl.multiple_of(i, 16)
            idx = sid_v[pl.ds(off, 16)]
            mask = (idx >= _A_LEN) & (idx < _T_OFF)
            rel = jnp.where(mask, idx - _A_LEN, 0)
            vals = plsc.load_gather(buf_b, [rel], mask=mask)
            prev = out_v[pl.ds(off, 16)]
            out_v[pl.ds(off, 16)] = jnp.where(mask, vals, prev)

        if r + 1 < _ROWS_PER_W:
            cp_b = pltpu.async_copy(
                tab_t.at[c + 1].at[pl.ds(_A_LEN, _B_LEN)], buf_b, sem_b
            )

        pltpu.sync_copy(out_v, out_t.at[c])


def kernel(sid, cropped_waveform, emb_table):
    del cropped_waveform
    tail_flat = emb_table[_T_OFF:, :].T.reshape(-1)
    out_t = _lane_gather_kernel(emb_table.T, tail_flat, sid.astype(jnp.int32))
    return out_t.T


# speedup vs baseline: 1.0615x; 1.0615x over previous
import functools

import jax
import jax.numpy as jnp
from jax import lax
from jax.experimental import pallas as pl
from jax.experimental.pallas import tpu as pltpu
from jax.experimental.pallas import tpu_sc as plsc

NUM_SPEAKER = 100000
EMB_DIM = 64
BATCH = 4096

_info = plsc.get_sparse_core_info()
_NC, _NS = _info.num_cores, _info.num_subcores
_NW = _NC * _NS
_ROWS_PER_W = EMB_DIM // _NW


@functools.partial(
    pl.kernel,
    mesh=plsc.VectorSubcoreMesh(core_axis_name="c", subcore_axis_name="s"),
    out_type=jax.ShapeDtypeStruct((EMB_DIM, BATCH), jnp.float32),
    scratch_types=[
        pltpu.VMEM((NUM_SPEAKER,), jnp.float32),
        pltpu.VMEM((BATCH,), jnp.int32),
        pltpu.VMEM((BATCH,), jnp.float32),
        pltpu.SemaphoreType.DMA,
    ],
    compiler_params=pltpu.CompilerParams(
        use_tc_tiling_on_sc=True, needs_layout_passes=False
    ),
)
def _lane_gather_kernel(tab_t, sid_hbm, out_t, row_v, sid_v, out_v, sem):
    wid = lax.axis_index("s") * _NC + lax.axis_index("c")
    c0 = wid * _ROWS_PER_W

    cp = pltpu.async_copy(tab_t.at[c0], row_v, sem)
    pltpu.sync_copy(sid_hbm, sid_v)

    for r in range(_ROWS_PER_W):
        cp.wait()

        @plsc.parallel_loop(0, BATCH, step=16, unroll=8)
        def _(i):
            off = pl.multiple_of(i, 16)
            idx = sid_v[pl.ds(off, 16)]
            out_v[pl.ds(off, 16)] = plsc.load_gather(row_v, [idx])
        pltpu.sync_copy(out_v, out_t.at[c0 + r])
        if r + 1 < _ROWS_PER_W:
            cp = pltpu.async_copy(tab_t.at[c0 + r + 1], row_v, sem)


def kernel(sid, cropped_waveform, emb_table):
    del cropped_waveform
    out_t = _lane_gather_kernel(emb_table.T, sid.astype(jnp.int32))
    return out_t.T
